# baseline (speedup 1.0000x reference)
import jax
import jax.numpy as jnp
from jax import lax
from jax.experimental import pallas as pl
from jax.experimental.pallas import tpu as pltpu
from jax.experimental.pallas import tpu_sc as plsc

B = 4
HO = WO = 192
CC = 192
OUT_B = HO * WO * CC
IN_B = OUT_B // 4
TOTAL_OUT = B * OUT_B
NS = 16
NWIN = 2
WIN = OUT_B // (2 * NWIN)
SHARE = IN_B // NS
WSHARE = WIN // NS
CHUNK = 2304
NCHUNK = SHARE // CHUNK
GROUPS = CHUNK // 16
PK = 2 * CHUNK
NZC = WSHARE // CHUNK


def _scatter_body(pkd_hbm, out_hbm, win_sh, pk0, pk1, off0, off1, val0, val1,
                  lsem0, lsem1, ssem0, ssem1, zsem):
    c = lax.axis_index("c")
    s = lax.axis_index("s")
    pk = (pk0, pk1)
    off = (off0, off1)
    val = (val0, val1)
    lsem = (lsem0, lsem1)
    ssem = (ssem0, ssem1)

    def load_src(b, ch):
        return pkd_hbm.at[pl.ds(2 * (b * IN_B + s * SHARE) + ch * PK, PK)]

    for b in range(B):
        for w in range(NWIN):
            wbase = c * (NWIN * WIN) + w * WIN

            pltpu.async_copy(load_src(b, 0), pk0, lsem0)
            pltpu.async_copy(load_src(b, 1), pk1, lsem1)

            def zfill(g, carry):
                val0[pl.ds(g * 16, 16)] = jnp.zeros((16,), jnp.float32)
                return carry

            lax.fori_loop(0, GROUPS, zfill, 0)

            def zissue(z, carry):
                pltpu.async_copy(
                    val0, win_sh.at[pl.ds(s * WSHARE + z * CHUNK, CHUNK)],
                    zsem)
                return carry

            lax.fori_loop(0, NZC, zissue, 0)

            def zdrain(z, carry):
                pltpu.make_async_copy(
                    val0, win_sh.at[pl.ds(s * WSHARE + z * CHUNK, CHUNK)],
                    zsem).wait()
                return carry

            lax.fori_loop(0, NZC, zdrain, 0)
            plsc.subcore_barrier()

            def chunk_pair(i, carry):
                sdmas = []
                for j in (0, 1):
                    ch = 2 * i + j
                    pltpu.make_async_copy(load_src(b, ch), pk[j],
                                          lsem[j]).wait()

                    def body(g, carry2):
                        iv = pk[j][pl.ds(g * 16, 16)]
                        uv = lax.bitcast_convert_type(
                            pk[j][pl.ds(CHUNK + g * 16, 16)], jnp.float32)
                        rel = iv - wbase
                        inm = (iv >= wbase) & (rel < WIN)
                        off[j][pl.ds(g * 16, 16)] = jnp.where(
                            inm, rel, iv & 0xFFFF)
                        val[j][pl.ds(g * 16, 16)] = jnp.where(
                            inm, uv, jnp.zeros((16,), jnp.float32))
                        return carry2

                    lax.fori_loop(0, GROUPS, body, 0)

                    @pl.when(ch + 2 < NCHUNK)
                    def _():
                        pltpu.async_copy(load_src(b, ch + 2), pk[j], lsem[j])

                    sdmas.append(
                        pltpu.async_copy(val[j], win_sh.at[off[j]],
                                         ssem[j], add=True))
                for d in sdmas:
                    d.wait()
                return carry

            lax.fori_loop(0, NCHUNK // 2, chunk_pair, 0)
            plsc.subcore_barrier()

            out_base = b * OUT_B + wbase + s * WSHARE
            pltpu.sync_copy(win_sh.at[pl.ds(s * WSHARE, WSHARE)],
                            out_hbm.at[pl.ds(out_base, WSHARE)])
            plsc.subcore_barrier()


def kernel(updates, mask):
    idx = mask.reshape(-1).astype(jnp.int32)
    upd = jax.lax.bitcast_convert_type(updates.reshape(-1), jnp.int32)
    packed = jnp.stack(
        [idx.reshape(-1, CHUNK), upd.reshape(-1, CHUNK)], axis=1).reshape(-1)
    mesh = plsc.VectorSubcoreMesh(core_axis_name="c", subcore_axis_name="s")
    run = pl.kernel(
        _scatter_body,
        mesh=mesh,
        out_type=jax.ShapeDtypeStruct((TOTAL_OUT,), jnp.float32),
        scratch_types=[
            pltpu.VMEM_SHARED((WIN,), jnp.float32),
            pltpu.VMEM((PK,), jnp.int32),
            pltpu.VMEM((PK,), jnp.int32),
            pltpu.VMEM((CHUNK,), jnp.int32),
            pltpu.VMEM((CHUNK,), jnp.int32),
            pltpu.VMEM((CHUNK,), jnp.float32),
            pltpu.VMEM((CHUNK,), jnp.float32),
            pltpu.SemaphoreType.DMA,
            pltpu.SemaphoreType.DMA,
            pltpu.SemaphoreType.DMA,
            pltpu.SemaphoreType.DMA,
            pltpu.SemaphoreType.DMA,
        ],
    )
    out = run(packed)
    return out.reshape(B, HO, WO, CC)
